# baseline (speedup 1.0000x reference)
import functools
import jax
import jax.numpy as jnp
from jax import lax
from jax.experimental import pallas as pl
from jax.experimental.pallas import tpu as pltpu
from jax.experimental.pallas import tpu_sc as plsc

_M, _N = 128, 32768
_NC, _NS, _L = 2, 16, 16
_NW = _NC * _NS
_RPW = _M // _NW
_CHUNK = 8192


def _sc_body(x_hbm, o_hbm, buf):
    wid = lax.axis_index("s") * _NC + lax.axis_index("c")
    r0 = wid * _RPW
    nv = _CHUNK // _L

    def do_chunk(ci, carries):
        pltpu.sync_copy(x_hbm.at[pl.ds(r0, _RPW), pl.ds(ci * _CHUNK, _CHUNK)],
                        buf)

        def body(i, cs):
            new = []
            for r in range(_RPW):
                v = buf[r, pl.ds(i * _L, _L)]
                s = plsc.cumsum(v) + cs[r]
                buf[r, pl.ds(i * _L, _L)] = s
                new.append(s[_L - 1])
            return tuple(new)

        carries = lax.fori_loop(0, nv, body, carries)
        pltpu.sync_copy(buf,
                        o_hbm.at[pl.ds(r0, _RPW), pl.ds(ci * _CHUNK, _CHUNK)])
        return carries

    zero = jnp.float32(0)
    carries = (zero,) * _RPW
    lax.fori_loop(0, _N // _CHUNK, do_chunk, carries)


def kernel(x):
    mesh = plsc.VectorSubcoreMesh(core_axis_name="c", subcore_axis_name="s")
    f = functools.partial(
        pl.kernel,
        mesh=mesh,
        out_type=jax.ShapeDtypeStruct((_M, _N), jnp.float32),
        scratch_types=[pltpu.VMEM((_RPW, _CHUNK), jnp.float32)],
        compiler_params=pltpu.CompilerParams(needs_layout_passes=False),
    )(_sc_body)
    return f(x)


# speedup vs baseline: 1.3694x; 1.3694x over previous
import functools
import jax
import jax.numpy as jnp
from jax import lax
from jax.experimental import pallas as pl
from jax.experimental.pallas import tpu as pltpu
from jax.experimental.pallas import tpu_sc as plsc

_M, _N = 128, 32768
_NC, _NS, _L = 2, 16, 16
_NW = _NC * _NS
_RPW = _M // _NW
_CHUNK = 8192
_UNROLL = 8


def _sc_body(x_hbm, o_hbm, buf):
    wid = lax.axis_index("s") * _NC + lax.axis_index("c")
    r0 = wid * _RPW

    def do_chunk(ci, carries):
        pltpu.sync_copy(x_hbm.at[pl.ds(r0, _RPW), pl.ds(ci * _CHUNK, _CHUNK)],
                        buf)

        def body(i, cs):
            cs = list(cs)
            base = i * (_L * _UNROLL)
            for u in range(_UNROLL):
                for r in range(_RPW):
                    v = buf[r, pl.ds(base + u * _L, _L)]
                    s = plsc.cumsum(v)
                    buf[r, pl.ds(base + u * _L, _L)] = s + cs[r]
                    cs[r] = cs[r] + s[_L - 1]
            return tuple(cs)

        carries = lax.fori_loop(0, _CHUNK // (_L * _UNROLL), body, carries)
        pltpu.sync_copy(buf,
                        o_hbm.at[pl.ds(r0, _RPW), pl.ds(ci * _CHUNK, _CHUNK)])
        return carries

    zero = jnp.float32(0)
    lax.fori_loop(0, _N // _CHUNK, do_chunk, (zero,) * _RPW)


def kernel(x):
    mesh = plsc.VectorSubcoreMesh(core_axis_name="c", subcore_axis_name="s")
    f = functools.partial(
        pl.kernel,
        mesh=mesh,
        out_type=jax.ShapeDtypeStruct((_M, _N), jnp.float32),
        scratch_types=[pltpu.VMEM((_RPW, _CHUNK), jnp.float32)],
        compiler_params=pltpu.CompilerParams(needs_layout_passes=False),
    )(_sc_body)
    return f(x)


# speedup vs baseline: 1.6059x; 1.1727x over previous
import functools
import jax
import jax.numpy as jnp
from jax import lax
from jax.experimental import pallas as pl
from jax.experimental.pallas import tpu as pltpu
from jax.experimental.pallas import tpu_sc as plsc

_M, _N = 128, 32768
_NC, _NS, _L = 2, 16, 16
_NW = _NC * _NS
_RPW = _M // _NW
_CHUNK = 4096
_UNROLL = 8


def _sc_body(x_hbm, o_hbm, buf0, buf1, si0, si1, so0, so1):
    bufs = (buf0, buf1)
    sins = (si0, si1)
    souts = (so0, so1)
    wid = lax.axis_index("s") * _NC + lax.axis_index("c")
    r0 = wid * _RPW
    nch = _N // _CHUNK

    def cols(ci):
        return pl.ds(ci * _CHUNK, _CHUNK)

    def compute(buf, carries):
        def body(i, cs):
            cs = list(cs)
            base = i * (_L * _UNROLL)
            for u in range(_UNROLL):
                for r in range(_RPW):
                    v = buf[r, pl.ds(base + u * _L, _L)]
                    s = plsc.cumsum(v)
                    buf[r, pl.ds(base + u * _L, _L)] = s + cs[r]
                    cs[r] = cs[r] + s[_L - 1]
            return tuple(cs)

        return lax.fori_loop(0, _CHUNK // (_L * _UNROLL), body, carries)

    descs_in = {}
    descs_out = {}
    descs_in[0] = pltpu.async_copy(
        x_hbm.at[pl.ds(r0, _RPW), cols(0)], bufs[0], sins[0])
    carries = (jnp.float32(0),) * _RPW
    for ci in range(nch):
        b = ci % 2
        descs_in[ci].wait()
        if ci + 1 < nch:
            if ci - 1 >= 0:
                descs_out[ci - 1].wait()
            descs_in[ci + 1] = pltpu.async_copy(
                x_hbm.at[pl.ds(r0, _RPW), cols(ci + 1)],
                bufs[1 - b], sins[1 - b])
        carries = compute(bufs[b], carries)
        descs_out[ci] = pltpu.async_copy(
            bufs[b], o_hbm.at[pl.ds(r0, _RPW), cols(ci)], souts[b])
    descs_out[nch - 2].wait()
    descs_out[nch - 1].wait()


def kernel(x):
    mesh = plsc.VectorSubcoreMesh(core_axis_name="c", subcore_axis_name="s")
    f = functools.partial(
        pl.kernel,
        mesh=mesh,
        out_type=jax.ShapeDtypeStruct((_M, _N), jnp.float32),
        scratch_types=[
            pltpu.VMEM((_RPW, _CHUNK), jnp.float32),
            pltpu.VMEM((_RPW, _CHUNK), jnp.float32),
            pltpu.SemaphoreType.DMA,
            pltpu.SemaphoreType.DMA,
            pltpu.SemaphoreType.DMA,
            pltpu.SemaphoreType.DMA,
        ],
        compiler_params=pltpu.CompilerParams(needs_layout_passes=False),
    )(_sc_body)
    return f(x)


# speedup vs baseline: 1.6705x; 1.0402x over previous
import functools
import jax
import jax.numpy as jnp
from jax import lax
from jax.experimental import pallas as pl
from jax.experimental.pallas import tpu as pltpu
from jax.experimental.pallas import tpu_sc as plsc

_M, _N = 128, 32768
_NC, _NS, _L = 2, 16, 16
_NW = _NC * _NS
_RPW = _M // _NW
_CHUNK = 8192
_UNROLL = 8


def _sc_body(x_hbm, o_hbm, buf0, buf1, si0, si1, so0, so1):
    bufs = (buf0, buf1)
    sins = (si0, si1)
    souts = (so0, so1)
    wid = lax.axis_index("s") * _NC + lax.axis_index("c")
    r0 = wid * _RPW
    nch = _N // _CHUNK

    def cols(ci):
        return pl.ds(ci * _CHUNK, _CHUNK)

    def compute(buf, carries):
        def body(i, cs):
            cs = list(cs)
            base = i * (_L * _UNROLL)
            for u in range(_UNROLL):
                for r in range(_RPW):
                    v = buf[r, pl.ds(base + u * _L, _L)]
                    s = plsc.cumsum(v)
                    buf[r, pl.ds(base + u * _L, _L)] = s + cs[r]
                    cs[r] = cs[r] + s[_L - 1]
            return tuple(cs)

        return lax.fori_loop(0, _CHUNK // (_L * _UNROLL), body, carries)

    descs_in = {}
    descs_out = {}
    descs_in[0] = pltpu.async_copy(
        x_hbm.at[pl.ds(r0, _RPW), cols(0)], bufs[0], sins[0])
    carries = (jnp.float32(0),) * _RPW
    for ci in range(nch):
        b = ci % 2
        descs_in[ci].wait()
        if ci + 1 < nch:
            if ci - 1 >= 0:
                descs_out[ci - 1].wait()
            descs_in[ci + 1] = pltpu.async_copy(
                x_hbm.at[pl.ds(r0, _RPW), cols(ci + 1)],
                bufs[1 - b], sins[1 - b])
        carries = compute(bufs[b], carries)
        descs_out[ci] = pltpu.async_copy(
            bufs[b], o_hbm.at[pl.ds(r0, _RPW), cols(ci)], souts[b])
    descs_out[nch - 2].wait()
    descs_out[nch - 1].wait()


def kernel(x):
    mesh = plsc.VectorSubcoreMesh(core_axis_name="c", subcore_axis_name="s")
    f = functools.partial(
        pl.kernel,
        mesh=mesh,
        out_type=jax.ShapeDtypeStruct((_M, _N), jnp.float32),
        scratch_types=[
            pltpu.VMEM((_RPW, _CHUNK), jnp.float32),
            pltpu.VMEM((_RPW, _CHUNK), jnp.float32),
            pltpu.SemaphoreType.DMA,
            pltpu.SemaphoreType.DMA,
            pltpu.SemaphoreType.DMA,
            pltpu.SemaphoreType.DMA,
        ],
        compiler_params=pltpu.CompilerParams(needs_layout_passes=False),
    )(_sc_body)
    return f(x)
